# Initial kernel scaffold; baseline (speedup 1.0000x reference)
import jax
import jax.numpy as jnp
from jax.experimental import pallas as pl


def kernel(x, edge_index, y, W1, b1, W2, b2):
    raise NotImplementedError("write your pallas kernel here")


# speedup vs baseline: 19.9454x; 19.9454x over previous
import functools

import jax
import jax.numpy as jnp
from jax import lax
from jax.experimental import pallas as pl
from jax.experimental.pallas import tpu as pltpu
from jax.experimental.pallas import tpu_sc as plsc

N = 10000
E = 320000
DIN = 128
DH = 64
HALF = 32
NP = 10240
NS = NP // 16
NV = NS // 16
CN = 128
NCH = 157
TEPT = NCH * CN
EP = 16 * TEPT
F32 = jnp.float32
I32 = jnp.int32

_Z16 = lambda: jnp.zeros((16,), F32)


def _mm_body(x_ref, w_ref, o_ref):
    o_ref[...] = jnp.dot(x_ref[...], w_ref[...], preferred_element_type=F32)


def _tc_matmul(xp, w):
    return pl.pallas_call(
        _mm_body,
        out_shape=jax.ShapeDtypeStruct((NP, 4 * DH), F32),
    )(xp, w)


def _sc_layer1(rowp, colp, z, b1):
    mesh = plsc.VectorSubcoreMesh(core_axis_name="c", subcore_axis_name="s")

    @functools.partial(
        pl.kernel,
        out_type=(
            jax.ShapeDtypeStruct((NP, DH), F32),
            jax.ShapeDtypeStruct((NP,), F32),
            jax.ShapeDtypeStruct((2 * NP, HALF), F32),
        ),
        mesh=mesh,
        compiler_params=pltpu.CompilerParams(use_tc_tiling_on_sc=False, needs_layout_passes=False),
        scratch_types=[
            pltpu.VMEM_SHARED((NP, HALF), F32),
            pltpu.VMEM_SHARED((16, NP), F32),
            pltpu.VMEM((NCH, CN), I32),
            pltpu.VMEM((NCH, CN), I32),
            pltpu.VMEM((CN, HALF), F32),
            pltpu.VMEM((NP,), F32),
            pltpu.VMEM((NS,), F32),
            pltpu.VMEM((NS,), F32),
            pltpu.VMEM((NS, HALF), F32),
            pltpu.VMEM((NS, HALF), F32),
            pltpu.VMEM((DH,), F32),
            pltpu.VMEM((NS,), F32),
            pltpu.SemaphoreType.DMA,
        ],
    )
    def k(rowp_h, colp_h, z_h, b1_h, h_h, dinv_h, g_h,
          acc, parts, rowb, colb, gbuf, dloc, psum, ptmp, zbuf, abuf,
          b1b, dvb, sem):
        c = lax.axis_index("c")
        s = lax.axis_index("s")
        n0 = s * NS
        cNP = c * NP

        pltpu.sync_copy(rowp_h.at[s], rowb)
        pltpu.sync_copy(colp_h.at[s], colb)
        pltpu.sync_copy(b1_h, b1b)

        def _off(j, _):
            for v in range(8):
                sl = pl.ds(v * 16, 16)
                rowb[j, sl] = rowb[j, sl] + cNP
            return 0
        lax.fori_loop(0, NCH, _off, 0)

        def _zd(i, _):
            dloc[pl.ds(i * 16, 16)] = _Z16()
            return 0
        lax.fori_loop(0, NP // 16, _zd, 0)

        ones = jnp.full((16,), 1.0, F32)

        def _hist(j, _):
            for v in range(8):
                idx = colb[j, pl.ds(v * 16, 16)]
                plsc.addupdate_scatter(dloc, [idx], ones)
            return 0
        lax.fori_loop(0, NCH, _hist, 0)

        pltpu.sync_copy(dloc, parts.at[s])
        plsc.subcore_barrier()

        def _zp(v, _):
            psum[pl.ds(v * 16, 16)] = _Z16()
            return 0
        lax.fori_loop(0, NV, _zp, 0)

        def _ap(p, _):
            pltpu.sync_copy(parts.at[p, pl.ds(n0, NS)], ptmp)

            def _add(v, _):
                sl = pl.ds(v * 16, 16)
                psum[sl] = psum[sl] + ptmp[sl]
                return 0
            lax.fori_loop(0, NV, _add, 0)
            return 0
        lax.fori_loop(0, 16, _ap, 0)

        def _dinv(v, _):
            sl = pl.ds(v * 16, 16)
            d = psum[sl]
            pos = d > 0.0
            dsafe = jnp.where(pos, d, 1.0)
            ib = plsc.bitcast(dsafe, I32)
            ib = jnp.int32(0x5F3759DF) - lax.shift_right_logical(ib, 1)
            y = plsc.bitcast(ib, F32)
            hd = dsafe * 0.5
            for _it in range(4):
                y = y * (1.5 - hd * y * y)
            dvb[sl] = jnp.where(pos, y, 0.0)
            return 0
        lax.fori_loop(0, NV, _dinv, 0)

        @pl.when(c == 0)
        def _():
            pltpu.sync_copy(dvb, dinv_h.at[pl.ds(n0, NS)])

        def _splat(ref, r):
            return plsc.load_gather(ref, [jnp.full((16,), r, I32)])

        def _za(r, _):
            for hh in range(2):
                abuf[r, pl.ds(hh * 16, 16)] = _Z16()
            return 0
        lax.fori_loop(0, NS, _za, 0)
        pltpu.sync_copy(abuf, acc.at[pl.ds(n0, NS)])

        pltpu.sync_copy(z_h.at[pl.ds(n0, NS), pl.ds(3 * DH + c * HALF, HALF)], zbuf)

        def _g3(r, _):
            dv = _splat(dvb, r)
            for hh in range(2):
                sl = (r, pl.ds(hh * 16, 16))
                zbuf[sl] = zbuf[sl] * dv
            return 0
        lax.fori_loop(0, NS, _g3, 0)
        pltpu.sync_copy(zbuf, g_h.at[pl.ds(cNP + n0, NS)])
        plsc.subcore_barrier()

        for kk in (2, 1, 0):
            def _edge(j, _):
                pltpu.async_copy(g_h.at[rowb.at[j]], gbuf, sem).wait()
                pltpu.sync_copy(gbuf, acc.at[colb.at[j]], add=True)
                return 0
            lax.fori_loop(0, NCH, _edge, 0)
            plsc.subcore_barrier()

            pltpu.sync_copy(acc.at[pl.ds(n0, NS)], abuf)
            pltpu.sync_copy(
                z_h.at[pl.ds(n0, NS), pl.ds(kk * DH + c * HALF, HALF)], zbuf)

            if kk > 0:
                def _comb(r, _):
                    dv = _splat(dvb, r)
                    dv2 = dv * dv
                    for hh in range(2):
                        sl = (r, pl.ds(hh * 16, 16))
                        zbuf[sl] = zbuf[sl] * dv + abuf[sl] * dv2
                    return 0
                lax.fori_loop(0, NS, _comb, 0)
                pltpu.sync_copy(zbuf, g_h.at[pl.ds(cNP + n0, NS)])
                lax.fori_loop(0, NS, _za, 0)
                pltpu.sync_copy(abuf, acc.at[pl.ds(n0, NS)])
                plsc.subcore_barrier()
            else:
                def _fin(r, _):
                    dv = _splat(dvb, r)
                    for hh in range(2):
                        sl = (r, pl.ds(hh * 16, 16))
                        v = zbuf[sl] + abuf[sl] * dv \
                            + b1b[pl.ds(c * HALF + hh * 16, 16)]
                        zbuf[sl] = jnp.maximum(v, 0.0)
                    return 0
                lax.fori_loop(0, NS, _fin, 0)
                pltpu.sync_copy(zbuf, h_h.at[pl.ds(n0, NS), pl.ds(c * HALF, HALF)])

    return k(rowp, colp, z, b1)


def _sc_layer2(rowp, colp, ht, w2, dinv, b2):
    mesh = plsc.VectorSubcoreMesh(core_axis_name="c", subcore_axis_name="s")

    @functools.partial(
        pl.kernel,
        out_type=jax.ShapeDtypeStruct((NP,), F32),
        mesh=mesh,
        compiler_params=pltpu.CompilerParams(use_tc_tiling_on_sc=False, needs_layout_passes=False),
        scratch_types=[
            pltpu.VMEM_SHARED((NP,), F32),
            pltpu.VMEM_SHARED((16, NP), F32),
            pltpu.VMEM((NCH, CN), I32),
            pltpu.VMEM((NCH, CN), I32),
            pltpu.VMEM((NP,), F32),
            pltpu.VMEM((NP,), F32),
            pltpu.VMEM((DH, NS), F32),
            pltpu.VMEM((4, NS), F32),
            pltpu.VMEM((NS,), F32),
            pltpu.VMEM((NS,), F32),
            pltpu.VMEM((NS,), F32),
            pltpu.VMEM((NS,), F32),
            pltpu.VMEM((4 * DH,), F32),
            pltpu.VMEM((16,), F32),
        ],
    )
    def k(rowp_h, colp_h, ht_h, w2_h, dinv_h, b2_h, out_h,
          t_sh, parts, rowb, colb, tloc, sloc, hst, ub,
          dvb, psum, ptmp, ttmp, w2v, b2v):
        c = lax.axis_index("c")
        s = lax.axis_index("s")
        n0 = s * NS

        @pl.when(c == 0)
        def _body():
            pltpu.sync_copy(rowp_h.at[s], rowb)
            pltpu.sync_copy(colp_h.at[s], colb)
            pltpu.sync_copy(w2_h, w2v)
            pltpu.sync_copy(b2_h, b2v)
            pltpu.sync_copy(dinv_h.at[pl.ds(n0, NS)], dvb)
            pltpu.sync_copy(ht_h.at[pl.ds(0, DH), pl.ds(n0, NS)], hst)

            def _zu(v, _):
                for kk in range(4):
                    ub[kk, pl.ds(v * 16, 16)] = _Z16()
                return 0
            lax.fori_loop(0, NV, _zu, 0)

            def _mm(d, _):
                w = [plsc.load_gather(w2v, [jnp.full((16,), kk * DH + d, I32)])
                     for kk in range(4)]

                def _mv(v, _):
                    sl = pl.ds(v * 16, 16)
                    hv = hst[d, sl]
                    for kk in range(4):
                        ub[kk, sl] = ub[kk, sl] + hv * w[kk]
                    return 0
                lax.fori_loop(0, NV, _mv, 0)
                return 0
            lax.fori_loop(0, DH, _mm, 0)

            def _t3(v, _):
                sl = pl.ds(v * 16, 16)
                ttmp[sl] = dvb[sl] * ub[3, sl]
                return 0
            lax.fori_loop(0, NV, _t3, 0)
            pltpu.sync_copy(ttmp, t_sh.at[pl.ds(n0, NS)])
            plsc.subcore_barrier()
            pltpu.sync_copy(t_sh, tloc)

            for kk in (2, 1, 0):
                def _zs(i, _):
                    sloc[pl.ds(i * 16, 16)] = _Z16()
                    return 0
                lax.fori_loop(0, NP // 16, _zs, 0)

                def _edge(j, _):
                    for v in range(8):
                        sl = pl.ds(v * 16, 16)
                        vals = plsc.load_gather(tloc, [rowb[j, sl]])
                        plsc.addupdate_scatter(sloc, [colb[j, sl]], vals)
                    return 0
                lax.fori_loop(0, NCH, _edge, 0)

                pltpu.sync_copy(sloc, parts.at[s])
                plsc.subcore_barrier()

                def _zp(v, _):
                    psum[pl.ds(v * 16, 16)] = _Z16()
                    return 0
                lax.fori_loop(0, NV, _zp, 0)

                def _ap(p, _):
                    pltpu.sync_copy(parts.at[p, pl.ds(n0, NS)], ptmp)

                    def _add(v, _):
                        sl = pl.ds(v * 16, 16)
                        psum[sl] = psum[sl] + ptmp[sl]
                        return 0
                    lax.fori_loop(0, NV, _add, 0)
                    return 0
                lax.fori_loop(0, 16, _ap, 0)

                if kk > 0:
                    def _cmb(v, _):
                        sl = pl.ds(v * 16, 16)
                        d = dvb[sl]
                        ttmp[sl] = d * ub[kk, sl] + d * d * psum[sl]
                        return 0
                    lax.fori_loop(0, NV, _cmb, 0)
                    pltpu.sync_copy(ttmp, t_sh.at[pl.ds(n0, NS)])
                    plsc.subcore_barrier()
                    pltpu.sync_copy(t_sh, tloc)
                else:
                    b2vec = plsc.load_gather(b2v, [jnp.zeros((16,), I32)])

                    def _fin(v, _):
                        sl = pl.ds(v * 16, 16)
                        ttmp[sl] = ub[0, sl] + dvb[sl] * psum[sl] + b2vec
                        return 0
                    lax.fori_loop(0, NV, _fin, 0)
                    pltpu.sync_copy(ttmp, out_h.at[pl.ds(n0, NS)])

    return k(rowp, colp, ht, w2, dinv, b2)


def kernel(x, edge_index, y, W1, b1, W2, b2):
    row = edge_index[0].astype(I32)
    col = edge_index[1].astype(I32)
    pad = EP - E
    rowp = jnp.concatenate([row, jnp.zeros((pad,), I32)]).reshape(16, NCH, CN)
    colp = jnp.concatenate([col, jnp.full((pad,), N, I32)]).reshape(16, NCH, CN)

    xp = jnp.concatenate([x, jnp.zeros((NP - N, DIN), F32)])
    w1m = W1.transpose(1, 0, 2).reshape(DIN, 4 * DH)
    z = _tc_matmul(xp, w1m)

    h, dinv, _g = _sc_layer1(rowp, colp, z, b1)

    ht = h.T
    w2f = W2[..., 0].reshape(-1)
    b2p = jnp.concatenate([b2, jnp.zeros((15,), F32)])
    outp = _sc_layer2(rowp, colp, ht, w2f, dinv, b2p)
    return outp[:N].reshape(N, 1)


# speedup vs baseline: 29.9009x; 1.4991x over previous
import functools

import jax
import jax.numpy as jnp
from jax import lax
from jax.experimental import pallas as pl
from jax.experimental.pallas import tpu as pltpu
from jax.experimental.pallas import tpu_sc as plsc

N = 10000
E = 320000
DIN = 128
DH = 64
HALF = 32
NP = 10240
NS = NP // 16
NV = NS // 16
CN = 128
NCH = 157
TEPT = NCH * CN
EP = 16 * TEPT
F32 = jnp.float32
I32 = jnp.int32

_Z16 = lambda: jnp.zeros((16,), F32)


def _mm_body(x_ref, w_ref, o_ref):
    o_ref[...] = jnp.dot(x_ref[...], w_ref[...], preferred_element_type=F32)


def _tc_matmul(xp, w):
    return pl.pallas_call(
        _mm_body,
        out_shape=jax.ShapeDtypeStruct((NP, 4 * DH), F32),
    )(xp, w)


def _sc_layer1(rowp, colp, z, b1):
    mesh = plsc.VectorSubcoreMesh(core_axis_name="c", subcore_axis_name="s")

    @functools.partial(
        pl.kernel,
        out_type=(
            jax.ShapeDtypeStruct((NP, DH), F32),
            jax.ShapeDtypeStruct((NP,), F32),
            jax.ShapeDtypeStruct((2 * NP, HALF), F32),
        ),
        mesh=mesh,
        compiler_params=pltpu.CompilerParams(use_tc_tiling_on_sc=False, needs_layout_passes=False),
        scratch_types=[
            pltpu.VMEM_SHARED((NP, HALF), F32),
            pltpu.VMEM_SHARED((NP,), F32),
            pltpu.VMEM((NCH, CN), I32),
            pltpu.VMEM((NCH, CN), I32),
            pltpu.VMEM((6, CN, HALF), F32),
            pltpu.VMEM((CN,), F32),
            pltpu.VMEM((NS,), F32),
            pltpu.VMEM((NS, HALF), F32),
            pltpu.VMEM((NS, HALF), F32),
            pltpu.VMEM((DH,), F32),
            pltpu.VMEM((NS,), F32),
            pltpu.SemaphoreType.DMA,
            pltpu.SemaphoreType.DMA,
        ],
    )
    def k(rowp_h, colp_h, z_h, b1_h, h_h, dinv_h, g_h,
          acc, degacc, rowb, colb, gbuf, onesb, psum, zbuf, abuf,
          b1b, dvb, gsem, ssem):
        c = lax.axis_index("c")
        s = lax.axis_index("s")
        n0 = s * NS
        cNP = c * NP

        pltpu.sync_copy(rowp_h.at[s], rowb)
        pltpu.sync_copy(colp_h.at[s], colb)
        pltpu.sync_copy(b1_h, b1b)

        def _off(j, _):
            for v in range(8):
                sl = pl.ds(v * 16, 16)
                rowb[j, sl] = rowb[j, sl] + cNP
            return 0
        lax.fori_loop(0, NCH, _off, 0)

        for v in range(CN // 16):
            onesb[pl.ds(v * 16, 16)] = jnp.full((16,), 1.0, F32)

        def _zp(v, _):
            psum[pl.ds(v * 16, 16)] = _Z16()
            return 0
        lax.fori_loop(0, NV, _zp, 0)
        pltpu.sync_copy(psum, degacc.at[pl.ds(n0, NS)])
        plsc.subcore_barrier()

        DB = 8

        def _dblk(j0, nb):
            sd = [pltpu.async_copy(onesb, degacc.at[colb.at[j0 + b]],
                                   ssem, add=True)
                  for b in range(nb)]
            for d in sd:
                d.wait()

        def _hblk(blk, _):
            _dblk(blk * DB, DB)
            return 0
        lax.fori_loop(0, NCH // DB, _hblk, 0)
        _dblk((NCH // DB) * DB, NCH - (NCH // DB) * DB)
        plsc.subcore_barrier()
        pltpu.sync_copy(degacc.at[pl.ds(n0, NS)], psum)

        def _dinv(v, _):
            sl = pl.ds(v * 16, 16)
            d = psum[sl]
            pos = d > 0.0
            dsafe = jnp.where(pos, d, 1.0)
            ib = plsc.bitcast(dsafe, I32)
            ib = jnp.int32(0x5F3759DF) - lax.shift_right_logical(ib, 1)
            y = plsc.bitcast(ib, F32)
            hd = dsafe * 0.5
            for _it in range(4):
                y = y * (1.5 - hd * y * y)
            dvb[sl] = jnp.where(pos, y, 0.0)
            return 0
        lax.fori_loop(0, NV, _dinv, 0)

        @pl.when(c == 0)
        def _():
            pltpu.sync_copy(dvb, dinv_h.at[pl.ds(n0, NS)])

        def _splat(ref, r):
            return plsc.load_gather(ref, [jnp.full((16,), r, I32)])

        def _za(r, _):
            for hh in range(2):
                abuf[r, pl.ds(hh * 16, 16)] = _Z16()
            return 0
        lax.fori_loop(0, NS, _za, 0)
        pltpu.sync_copy(abuf, acc.at[pl.ds(n0, NS)])

        pltpu.sync_copy(z_h.at[pl.ds(n0, NS), pl.ds(3 * DH + c * HALF, HALF)], zbuf)

        def _g3(r, _):
            dv = _splat(dvb, r)
            for hh in range(2):
                sl = (r, pl.ds(hh * 16, 16))
                zbuf[sl] = zbuf[sl] * dv
            return 0
        lax.fori_loop(0, NS, _g3, 0)
        pltpu.sync_copy(zbuf, g_h.at[pl.ds(cNP + n0, NS)])
        plsc.subcore_barrier()

        NB = 6

        def _edge_block(j0, nb):
            gd = [pltpu.async_copy(g_h.at[rowb.at[j0 + b]], gbuf.at[b], gsem)
                  for b in range(nb)]
            for d in gd:
                d.wait()
            sd = [pltpu.async_copy(gbuf.at[b], acc.at[colb.at[j0 + b]],
                                   ssem, add=True)
                  for b in range(nb)]
            for d in sd:
                d.wait()

        for kk in (2, 1, 0):
            def _blk(blk, _):
                _edge_block(blk * NB, NB)
                return 0
            lax.fori_loop(0, NCH // NB, _blk, 0)
            _edge_block((NCH // NB) * NB, NCH - (NCH // NB) * NB)
            plsc.subcore_barrier()

            pltpu.sync_copy(acc.at[pl.ds(n0, NS)], abuf)
            pltpu.sync_copy(
                z_h.at[pl.ds(n0, NS), pl.ds(kk * DH + c * HALF, HALF)], zbuf)

            if kk > 0:
                def _comb(r, _):
                    dv = _splat(dvb, r)
                    dv2 = dv * dv
                    for hh in range(2):
                        sl = (r, pl.ds(hh * 16, 16))
                        zbuf[sl] = zbuf[sl] * dv + abuf[sl] * dv2
                    return 0
                lax.fori_loop(0, NS, _comb, 0)
                pltpu.sync_copy(zbuf, g_h.at[pl.ds(cNP + n0, NS)])
                lax.fori_loop(0, NS, _za, 0)
                pltpu.sync_copy(abuf, acc.at[pl.ds(n0, NS)])
                plsc.subcore_barrier()
            else:
                def _fin(r, _):
                    dv = _splat(dvb, r)
                    for hh in range(2):
                        sl = (r, pl.ds(hh * 16, 16))
                        v = zbuf[sl] + abuf[sl] * dv \
                            + b1b[pl.ds(c * HALF + hh * 16, 16)]
                        zbuf[sl] = jnp.maximum(v, 0.0)
                    return 0
                lax.fori_loop(0, NS, _fin, 0)
                pltpu.sync_copy(zbuf, h_h.at[pl.ds(n0, NS), pl.ds(c * HALF, HALF)])

    return k(rowp, colp, z, b1)


def _sc_layer2(rowp, colp, ht, w2, dinv, b2):
    mesh = plsc.VectorSubcoreMesh(core_axis_name="c", subcore_axis_name="s")

    @functools.partial(
        pl.kernel,
        out_type=jax.ShapeDtypeStruct((NP,), F32),
        mesh=mesh,
        compiler_params=pltpu.CompilerParams(use_tc_tiling_on_sc=False, needs_layout_passes=False),
        scratch_types=[
            pltpu.VMEM_SHARED((NP,), F32),
            pltpu.VMEM_SHARED((16, NP), F32),
            pltpu.VMEM((NCH, CN), I32),
            pltpu.VMEM((NCH, CN), I32),
            pltpu.VMEM((NP,), F32),
            pltpu.VMEM((NP,), F32),
            pltpu.VMEM((DH, NS), F32),
            pltpu.VMEM((4, NS), F32),
            pltpu.VMEM((NS,), F32),
            pltpu.VMEM((NS,), F32),
            pltpu.VMEM((NS,), F32),
            pltpu.VMEM((NS,), F32),
            pltpu.VMEM((4 * DH,), F32),
            pltpu.VMEM((16,), F32),
        ],
    )
    def k(rowp_h, colp_h, ht_h, w2_h, dinv_h, b2_h, out_h,
          t_sh, parts, rowb, colb, tloc, sloc, hst, ub,
          dvb, psum, ptmp, ttmp, w2v, b2v):
        c = lax.axis_index("c")
        s = lax.axis_index("s")
        n0 = s * NS

        @pl.when(c == 0)
        def _body():
            pltpu.sync_copy(rowp_h.at[s], rowb)
            pltpu.sync_copy(colp_h.at[s], colb)
            pltpu.sync_copy(w2_h, w2v)
            pltpu.sync_copy(b2_h, b2v)
            pltpu.sync_copy(dinv_h.at[pl.ds(n0, NS)], dvb)
            pltpu.sync_copy(ht_h.at[pl.ds(0, DH), pl.ds(n0, NS)], hst)

            def _zu(v, _):
                for kk in range(4):
                    ub[kk, pl.ds(v * 16, 16)] = _Z16()
                return 0
            lax.fori_loop(0, NV, _zu, 0)

            def _mm(d, _):
                w = [plsc.load_gather(w2v, [jnp.full((16,), kk * DH + d, I32)])
                     for kk in range(4)]

                def _mv(v, _):
                    sl = pl.ds(v * 16, 16)
                    hv = hst[d, sl]
                    for kk in range(4):
                        ub[kk, sl] = ub[kk, sl] + hv * w[kk]
                    return 0
                lax.fori_loop(0, NV, _mv, 0)
                return 0
            lax.fori_loop(0, DH, _mm, 0)

            def _t3(v, _):
                sl = pl.ds(v * 16, 16)
                ttmp[sl] = dvb[sl] * ub[3, sl]
                return 0
            lax.fori_loop(0, NV, _t3, 0)
            pltpu.sync_copy(ttmp, t_sh.at[pl.ds(n0, NS)])
            plsc.subcore_barrier()
            pltpu.sync_copy(t_sh, tloc)

            for kk in (2, 1, 0):
                def _zs(i, _):
                    sloc[pl.ds(i * 16, 16)] = _Z16()
                    return 0
                lax.fori_loop(0, NP // 16, _zs, 0)

                def _edge(j, _):
                    for v in range(8):
                        sl = pl.ds(v * 16, 16)
                        vals = plsc.load_gather(tloc, [rowb[j, sl]])
                        plsc.addupdate_scatter(sloc, [colb[j, sl]], vals)
                    return 0
                lax.fori_loop(0, NCH, _edge, 0)

                pltpu.sync_copy(sloc, parts.at[s])
                plsc.subcore_barrier()

                def _zp(v, _):
                    psum[pl.ds(v * 16, 16)] = _Z16()
                    return 0
                lax.fori_loop(0, NV, _zp, 0)

                def _ap(p, _):
                    pltpu.sync_copy(parts.at[p, pl.ds(n0, NS)], ptmp)

                    def _add(v, _):
                        sl = pl.ds(v * 16, 16)
                        psum[sl] = psum[sl] + ptmp[sl]
                        return 0
                    lax.fori_loop(0, NV, _add, 0)
                    return 0
                lax.fori_loop(0, 16, _ap, 0)

                if kk > 0:
                    def _cmb(v, _):
                        sl = pl.ds(v * 16, 16)
                        d = dvb[sl]
                        ttmp[sl] = d * ub[kk, sl] + d * d * psum[sl]
                        return 0
                    lax.fori_loop(0, NV, _cmb, 0)
                    pltpu.sync_copy(ttmp, t_sh.at[pl.ds(n0, NS)])
                    plsc.subcore_barrier()
                    pltpu.sync_copy(t_sh, tloc)
                else:
                    b2vec = plsc.load_gather(b2v, [jnp.zeros((16,), I32)])

                    def _fin(v, _):
                        sl = pl.ds(v * 16, 16)
                        ttmp[sl] = ub[0, sl] + dvb[sl] * psum[sl] + b2vec
                        return 0
                    lax.fori_loop(0, NV, _fin, 0)
                    pltpu.sync_copy(ttmp, out_h.at[pl.ds(n0, NS)])

    return k(rowp, colp, ht, w2, dinv, b2)


def kernel(x, edge_index, y, W1, b1, W2, b2):
    row = edge_index[0].astype(I32)
    col = edge_index[1].astype(I32)
    pad = EP - E
    rowp = jnp.concatenate([row, jnp.zeros((pad,), I32)]).reshape(16, NCH, CN)
    colp = jnp.concatenate([col, jnp.full((pad,), N, I32)]).reshape(16, NCH, CN)

    xp = jnp.concatenate([x, jnp.zeros((NP - N, DIN), F32)])
    w1m = W1.transpose(1, 0, 2).reshape(DIN, 4 * DH)
    z = _tc_matmul(xp, w1m)

    h, dinv, _g = _sc_layer1(rowp, colp, z, b1)

    ht = h.T
    w2f = W2[..., 0].reshape(-1)
    b2p = jnp.concatenate([b2, jnp.zeros((15,), F32)])
    outp = _sc_layer2(rowp, colp, ht, w2f, dinv, b2p)
    return outp[:N].reshape(N, 1)
